# baseline (device time: 23836 ns/iter reference)
import jax
import jax.numpy as jnp
from jax import lax
from jax.experimental import pallas as pl
from jax.experimental.pallas import tpu as pltpu

CHUNKS = (64, 192, 256, 192, 128, 96, 64, 32)
N_CHUNKS = len(CHUNKS)
OFFSETS = tuple(sum(CHUNKS[:i]) for i in range(N_CHUNKS))


def kernel(x):
    _, m, n = x.shape
    assert m == sum(CHUNKS)

    def body(x_ref, out_ref, xsend, xrecv, ysend, yrecv,
             x_send_sems, x_recv_sems, y_send_sems, y_recv_sems):
        my_x = lax.axis_index("x")
        my_y = lax.axis_index("y")
        x_partner = (1 - my_x, my_y)
        y_partner = (my_x, 1 - my_y)
        own_col = pl.ds(my_y * n, n)
        oth_col = pl.ds((1 - my_y) * n, n)

        def rows(c):
            return pl.ds(OFFSETS[c], CHUNKS[c])

        barrier = pltpu.get_barrier_semaphore()
        for nbr in (x_partner, y_partner):
            pl.semaphore_signal(barrier, inc=1, device_id=nbr,
                                device_id_type=pl.DeviceIdType.MESH)
        xsend[rows(0)] = x_ref[0, rows(0), :].astype(jnp.bfloat16)
        pl.semaphore_wait(barrier, 2)

        x_rdmas = []
        for c in range(N_CHUNKS):
            if c > 0:
                xsend[rows(c)] = x_ref[0, rows(c), :].astype(jnp.bfloat16)
            rdma = pltpu.make_async_remote_copy(
                src_ref=xsend.at[rows(c)], dst_ref=xrecv.at[rows(c)],
                send_sem=x_send_sems.at[c], recv_sem=x_recv_sems.at[c],
                device_id=x_partner, device_id_type=pl.DeviceIdType.MESH)
            rdma.start()
            x_rdmas.append(rdma)

        y_rdmas = []
        for c in range(N_CHUNKS):
            x_rdmas[c].wait_recv()
            ysend[rows(c)] = xsend[rows(c)] + xrecv[rows(c)]
            rdma = pltpu.make_async_remote_copy(
                src_ref=ysend.at[rows(c)], dst_ref=yrecv.at[rows(c)],
                send_sem=y_send_sems.at[c], recv_sem=y_recv_sems.at[c],
                device_id=y_partner, device_id_type=pl.DeviceIdType.MESH)
            rdma.start()
            y_rdmas.append(rdma)

        for c in range(N_CHUNKS):
            out_ref[rows(c), own_col] = ysend[rows(c)].astype(jnp.float32)

        for c in range(N_CHUNKS):
            y_rdmas[c].wait_recv()
            out_ref[rows(c), oth_col] = yrecv[rows(c)].astype(jnp.float32)

        for c in range(N_CHUNKS):
            x_rdmas[c].wait_send()
            y_rdmas[c].wait_send()

    return pl.pallas_call(
        body,
        out_shape=jax.ShapeDtypeStruct((m, 2 * n), jnp.float32),
        in_specs=[pl.BlockSpec(memory_space=pltpu.VMEM)],
        out_specs=pl.BlockSpec(memory_space=pltpu.VMEM),
        scratch_shapes=[
            pltpu.VMEM((m, n), jnp.bfloat16),
            pltpu.VMEM((m, n), jnp.bfloat16),
            pltpu.VMEM((m, n), jnp.bfloat16),
            pltpu.VMEM((m, n), jnp.bfloat16),
            pltpu.SemaphoreType.DMA((N_CHUNKS,)),
            pltpu.SemaphoreType.DMA((N_CHUNKS,)),
            pltpu.SemaphoreType.DMA((N_CHUNKS,)),
            pltpu.SemaphoreType.DMA((N_CHUNKS,)),
        ],
        compiler_params=pltpu.CompilerParams(collective_id=0),
    )(x)
